# Initial kernel scaffold; baseline (speedup 1.0000x reference)
import jax
import jax.numpy as jnp
from jax import lax
from jax.experimental import pallas as pl
from jax.experimental.pallas import tpu as pltpu
from jax.experimental.pallas import tpu_sc as plsc

N = 10000
E = 320000
D = 128
C = 40

NC = 2
NS = 16
NW = NC * NS

NPAD = 10240
K = 128
EPT = 10240
NCHUNK = EPT // K
EPAD = EPT * NW
ROWS_PER_TILE = NPAD // NS

_BN_C = 1.0 / (1.0 + 1e-5) ** 0.5


def _sc_mesh():
    return plsc.VectorSubcoreMesh(core_axis_name="c", subcore_axis_name="s")


def _deg_body(dst_hbm, onehot_hbm, zero16_hbm, out_hbm,
              dstv, onehot_v, acc, sem):
    cid = lax.axis_index("c")
    sid = lax.axis_index("s")

    row0 = sid * ROWS_PER_TILE
    pltpu.sync_copy(zero16_hbm, acc.at[pl.ds(row0, ROWS_PER_TILE)])
    pltpu.sync_copy(onehot_hbm, onehot_v)
    plsc.subcore_barrier()

    base0 = (cid * NS + sid) * EPT

    def chunk(i, carry):
        base = base0 + i * K
        pltpu.sync_copy(dst_hbm.at[pl.ds(base, K)], dstv)
        pltpu.sync_copy(onehot_v, acc.at[dstv], add=True)
        return carry

    lax.fori_loop(0, NCHUNK, chunk, 0)
    plsc.subcore_barrier()

    pltpu.sync_copy(acc.at[pl.ds(row0, ROWS_PER_TILE)],
                    out_hbm.at[cid, pl.ds(row0, ROWS_PER_TILE)])


def _scatter_body(hp_hbm, src_hbm, dst_hbm, zero128_hbm, out_hbm,
                  srcv, dstv, rows, acc, sem):
    cid = lax.axis_index("c")
    sid = lax.axis_index("s")

    row0 = sid * ROWS_PER_TILE
    for k in range(ROWS_PER_TILE // 128):
        pltpu.sync_copy(zero128_hbm, acc.at[pl.ds(row0 + k * 128, 128)])
    plsc.subcore_barrier()

    base0 = (cid * NS + sid) * EPT

    def chunk(i, carry):
        base = base0 + i * K
        pltpu.sync_copy(src_hbm.at[pl.ds(base, K)], srcv)
        pltpu.sync_copy(dst_hbm.at[pl.ds(base, K)], dstv)
        pltpu.async_copy(hp_hbm.at[srcv], rows, sem).wait()
        pltpu.sync_copy(rows, acc.at[dstv], add=True)
        return carry

    lax.fori_loop(0, NCHUNK, chunk, 0)
    plsc.subcore_barrier()

    pltpu.sync_copy(acc.at[pl.ds(row0, ROWS_PER_TILE)],
                    out_hbm.at[cid, pl.ds(row0, ROWS_PER_TILE)])


def _make_deg():
    return pl.kernel(
        _deg_body,
        out_type=jax.ShapeDtypeStruct((NC, NPAD, 16), jnp.float32),
        mesh=_sc_mesh(),
        scratch_types=[
            pltpu.VMEM((K,), jnp.int32),
            pltpu.VMEM((K, 16), jnp.float32),
            pltpu.VMEM_SHARED((NPAD, 16), jnp.float32),
            pltpu.SemaphoreType.DMA,
        ],
    )


def _make_scatter():
    return pl.kernel(
        _scatter_body,
        out_type=jax.ShapeDtypeStruct((NC, NPAD, D), jnp.float32),
        mesh=_sc_mesh(),
        scratch_types=[
            pltpu.VMEM((K,), jnp.int32),
            pltpu.VMEM((K,), jnp.int32),
            pltpu.VMEM((K, D), jnp.float32),
            pltpu.VMEM_SHARED((NPAD, D), jnp.float32),
            pltpu.SemaphoreType.DMA,
        ],
    )


def _dinv_body(d0_ref, d1_ref, out_ref):
    deg = d0_ref[:, 0:1] + d1_ref[:, 0:1] + 1.0
    out_ref[...] = lax.rsqrt(deg)


def _dinv_call(deg_parts):
    return pl.pallas_call(
        _dinv_body,
        grid=(NPAD // 128,),
        in_specs=[
            pl.BlockSpec((128, 16), lambda i: (i, 0)),
            pl.BlockSpec((128, 16), lambda i: (i, 0)),
        ],
        out_specs=pl.BlockSpec((128, 1), lambda i: (i, 0)),
        out_shape=jax.ShapeDtypeStruct((NPAD, 1), jnp.float32),
    )(deg_parts[0], deg_parts[1])


def _mm_scale_body(x_ref, w_ref, dinv_ref, out_ref):
    h = jnp.dot(x_ref[...], w_ref[...], preferred_element_type=jnp.float32)
    out_ref[...] = h * dinv_ref[...]


def _mm_scale_call(x, w, dinv):
    return pl.pallas_call(
        _mm_scale_body,
        grid=(NPAD // 128,),
        in_specs=[
            pl.BlockSpec((128, D), lambda i: (i, 0)),
            pl.BlockSpec((D, D), lambda i: (0, 0)),
            pl.BlockSpec((128, 1), lambda i: (i, 0)),
        ],
        out_specs=pl.BlockSpec((128, D), lambda i: (i, 0)),
        out_shape=jax.ShapeDtypeStruct((NPAD, D), jnp.float32),
    )(x, w, dinv)


def _combine_mm_body(s_ref, hp_ref, dinv_ref, b_ref, g_ref, be_ref, w_ref,
                     out_ref):
    y = dinv_ref[...] * (s_ref[0] + s_ref[1] + hp_ref[...]) + b_ref[...]
    xn = jnp.maximum(y * (g_ref[...] * _BN_C) + be_ref[...], 0.0)
    h = jnp.dot(xn, w_ref[...], preferred_element_type=jnp.float32)
    out_ref[...] = h * dinv_ref[...]


def _combine_mm_call(s, hp, dinv, b, g, be, w):
    return pl.pallas_call(
        _combine_mm_body,
        grid=(NPAD // 128,),
        in_specs=[
            pl.BlockSpec((NC, 128, D), lambda i: (0, i, 0)),
            pl.BlockSpec((128, D), lambda i: (i, 0)),
            pl.BlockSpec((128, 1), lambda i: (i, 0)),
            pl.BlockSpec((1, D), lambda i: (0, 0)),
            pl.BlockSpec((1, D), lambda i: (0, 0)),
            pl.BlockSpec((1, D), lambda i: (0, 0)),
            pl.BlockSpec((D, D), lambda i: (0, 0)),
        ],
        out_specs=pl.BlockSpec((128, D), lambda i: (i, 0)),
        out_shape=jax.ShapeDtypeStruct((NPAD, D), jnp.float32),
    )(s, hp, dinv, b, g, be, w)


def _final_body(s_ref, hp_ref, dinv_ref, b_ref, g_ref, be_ref, wfc_ref,
                bfc_ref, logp_ref, z_ref):
    y = dinv_ref[...] * (s_ref[0] + s_ref[1] + hp_ref[...]) + b_ref[...]
    xn = jnp.maximum(y * (g_ref[...] * _BN_C) + be_ref[...], 0.0)
    z = jnp.maximum(
        jnp.dot(xn, wfc_ref[...], preferred_element_type=jnp.float32)
        + bfc_ref[...], 0.0)
    col = lax.broadcasted_iota(jnp.int32, (128, D), 1)
    valid = col < C
    zm = jnp.where(valid, z, -jnp.inf)
    m = jnp.max(zm, axis=1, keepdims=True)
    ssum = jnp.sum(jnp.where(valid, jnp.exp(zm - m), 0.0), axis=1,
                   keepdims=True)
    logp_ref[...] = z - (m + jnp.log(ssum))
    z_ref[...] = z


def _final_call(s, hp, dinv, b, g, be, wfc_pad, bfc_pad):
    return pl.pallas_call(
        _final_body,
        grid=(NPAD // 128,),
        in_specs=[
            pl.BlockSpec((NC, 128, D), lambda i: (0, i, 0)),
            pl.BlockSpec((128, D), lambda i: (i, 0)),
            pl.BlockSpec((128, 1), lambda i: (i, 0)),
            pl.BlockSpec((1, D), lambda i: (0, 0)),
            pl.BlockSpec((1, D), lambda i: (0, 0)),
            pl.BlockSpec((1, D), lambda i: (0, 0)),
            pl.BlockSpec((D, D), lambda i: (0, 0)),
            pl.BlockSpec((1, D), lambda i: (0, 0)),
        ],
        out_specs=[
            pl.BlockSpec((128, D), lambda i: (i, 0)),
            pl.BlockSpec((128, D), lambda i: (i, 0)),
        ],
        out_shape=[
            jax.ShapeDtypeStruct((NPAD, D), jnp.float32),
            jax.ShapeDtypeStruct((NPAD, D), jnp.float32),
        ],
    )(s, hp, dinv, b, g, be, wfc_pad, bfc_pad)


def kernel(x, edge_index, W1, b1, g1, be1, W2, b2, g2, be2, W3, b3, g3, be3,
           Wfc, bfc):
    f32 = jnp.float32
    xpad = jnp.zeros((NPAD, D), f32).at[:N].set(x)
    pad_idx = jnp.full((EPAD - E,), NPAD - 1, jnp.int32)
    src = jnp.concatenate([edge_index[0], pad_idx])
    dst = jnp.concatenate([edge_index[1], pad_idx])

    onehot = jnp.zeros((K, 16), f32).at[:, 0].set(1.0)
    zero16 = jnp.zeros((ROWS_PER_TILE, 16), f32)
    zero128 = jnp.zeros((128, D), f32)
    wfc_pad = jnp.zeros((D, D), f32).at[:, :C].set(Wfc)
    bfc_pad = jnp.zeros((1, D), f32).at[0, :C].set(bfc)

    deg_parts = _make_deg()(dst, onehot, zero16)
    dinv = _dinv_call(deg_parts)

    scatter = _make_scatter()

    hp = _mm_scale_call(xpad, W1, dinv)
    s = scatter(hp, src, dst, zero128)
    hp = _combine_mm_call(s, hp, dinv, b1.reshape(1, D), g1.reshape(1, D),
                          be1.reshape(1, D), W2)
    s = scatter(hp, src, dst, zero128)
    hp = _combine_mm_call(s, hp, dinv, b2.reshape(1, D), g2.reshape(1, D),
                          be2.reshape(1, D), W3)
    s = scatter(hp, src, dst, zero128)
    logp, z = _final_call(s, hp, dinv, b3.reshape(1, D), g3.reshape(1, D),
                          be3.reshape(1, D), wfc_pad, bfc_pad)

    return (logp[:N, :C], z[:N, :C])


# speedup vs baseline: 4.0404x; 4.0404x over previous
import jax
import jax.numpy as jnp
from jax import lax
from jax.experimental import pallas as pl
from jax.experimental.pallas import tpu as pltpu
from jax.experimental.pallas import tpu_sc as plsc

N = 10000
E = 320000
D = 128
C = 40

NC = 2
NS = 16
NW = NC * NS

NPAD = 10240
K = 128
EPT = 10240
NCHUNK = EPT // K
EPAD = EPT * NW
ROWS_PER_TILE = NPAD // NS

_BN_C = 1.0 / (1.0 + 1e-5) ** 0.5


def _sc_mesh():
    return plsc.VectorSubcoreMesh(core_axis_name="c", subcore_axis_name="s")


def _scatter_body(hp_hbm, src_hbm, dst_hbm, zero128_hbm, out_hbm,
                  srcv, dstv, rows, acc, sem):
    cid = lax.axis_index("c")
    sid = lax.axis_index("s")

    row0 = sid * ROWS_PER_TILE
    for k in range(ROWS_PER_TILE // 128):
        pltpu.sync_copy(zero128_hbm, acc.at[pl.ds(row0 + k * 128, 128)])
    plsc.subcore_barrier()

    base0 = (cid * NS + sid) * EPT

    def chunk(i, carry):
        base = base0 + i * K
        pltpu.sync_copy(src_hbm.at[pl.ds(base, K)], srcv)
        pltpu.sync_copy(dst_hbm.at[pl.ds(base, K)], dstv)
        pltpu.async_copy(hp_hbm.at[srcv], rows, sem).wait()
        pltpu.sync_copy(rows, acc.at[dstv], add=True)
        return carry

    lax.fori_loop(0, NCHUNK, chunk, 0)
    plsc.subcore_barrier()

    pltpu.sync_copy(acc.at[pl.ds(row0, ROWS_PER_TILE)],
                    out_hbm.at[cid, pl.ds(row0, ROWS_PER_TILE)])


def _make_scatter():
    return pl.kernel(
        _scatter_body,
        out_type=jax.ShapeDtypeStruct((NC, NPAD, D), jnp.float32),
        mesh=_sc_mesh(),
        scratch_types=[
            pltpu.VMEM((K,), jnp.int32),
            pltpu.VMEM((K,), jnp.int32),
            pltpu.VMEM((K, D), jnp.float32),
            pltpu.VMEM_SHARED((NPAD, D), jnp.float32),
            pltpu.SemaphoreType.DMA,
        ],
    )


def _dinv_body(d0_ref, d1_ref, out_ref):
    deg = d0_ref[:, 0:1] + d1_ref[:, 0:1] + 1.0
    out_ref[...] = lax.rsqrt(deg)


def _dinv_call(deg_parts):
    return pl.pallas_call(
        _dinv_body,
        grid=(NPAD // 128,),
        in_specs=[
            pl.BlockSpec((128, D), lambda i: (i, 0)),
            pl.BlockSpec((128, D), lambda i: (i, 0)),
        ],
        out_specs=pl.BlockSpec((128, 1), lambda i: (i, 0)),
        out_shape=jax.ShapeDtypeStruct((NPAD, 1), jnp.float32),
    )(deg_parts[0], deg_parts[1])


def _mm_scale_body(x_ref, w_ref, dinv_ref, out_ref):
    h = jnp.dot(x_ref[...], w_ref[...], preferred_element_type=jnp.float32)
    out_ref[...] = h * dinv_ref[...]


def _mm_scale_call(x, w, dinv):
    return pl.pallas_call(
        _mm_scale_body,
        grid=(NPAD // 128,),
        in_specs=[
            pl.BlockSpec((128, D), lambda i: (i, 0)),
            pl.BlockSpec((D, D), lambda i: (0, 0)),
            pl.BlockSpec((128, 1), lambda i: (i, 0)),
        ],
        out_specs=pl.BlockSpec((128, D), lambda i: (i, 0)),
        out_shape=jax.ShapeDtypeStruct((NPAD, D), jnp.float32),
    )(x, w, dinv)


def _combine_mm_body(s_ref, hp_ref, dinv_ref, b_ref, g_ref, be_ref, w_ref,
                     out_ref):
    y = dinv_ref[...] * (s_ref[0] + s_ref[1] + hp_ref[...]) + b_ref[...]
    xn = jnp.maximum(y * (g_ref[...] * _BN_C) + be_ref[...], 0.0)
    h = jnp.dot(xn, w_ref[...], preferred_element_type=jnp.float32)
    out_ref[...] = h * dinv_ref[...]


def _combine_mm_call(s, hp, dinv, b, g, be, w):
    return pl.pallas_call(
        _combine_mm_body,
        grid=(NPAD // 128,),
        in_specs=[
            pl.BlockSpec((NC, 128, D), lambda i: (0, i, 0)),
            pl.BlockSpec((128, D), lambda i: (i, 0)),
            pl.BlockSpec((128, 1), lambda i: (i, 0)),
            pl.BlockSpec((1, D), lambda i: (0, 0)),
            pl.BlockSpec((1, D), lambda i: (0, 0)),
            pl.BlockSpec((1, D), lambda i: (0, 0)),
            pl.BlockSpec((D, D), lambda i: (0, 0)),
        ],
        out_specs=pl.BlockSpec((128, D), lambda i: (i, 0)),
        out_shape=jax.ShapeDtypeStruct((NPAD, D), jnp.float32),
    )(s, hp, dinv, b, g, be, w)


def _final_body(s_ref, hp_ref, dinv_ref, b_ref, g_ref, be_ref, wfc_ref,
                bfc_ref, logp_ref, z_ref):
    y = dinv_ref[...] * (s_ref[0] + s_ref[1] + hp_ref[...]) + b_ref[...]
    xn = jnp.maximum(y * (g_ref[...] * _BN_C) + be_ref[...], 0.0)
    z = jnp.maximum(
        jnp.dot(xn, wfc_ref[...], preferred_element_type=jnp.float32)
        + bfc_ref[...], 0.0)
    col = lax.broadcasted_iota(jnp.int32, (128, D), 1)
    valid = col < C
    zm = jnp.where(valid, z, -jnp.inf)
    m = jnp.max(zm, axis=1, keepdims=True)
    ssum = jnp.sum(jnp.where(valid, jnp.exp(zm - m), 0.0), axis=1,
                   keepdims=True)
    logp_ref[...] = z - (m + jnp.log(ssum))
    z_ref[...] = z


def _final_call(s, hp, dinv, b, g, be, wfc_pad, bfc_pad):
    return pl.pallas_call(
        _final_body,
        grid=(NPAD // 128,),
        in_specs=[
            pl.BlockSpec((NC, 128, D), lambda i: (0, i, 0)),
            pl.BlockSpec((128, D), lambda i: (i, 0)),
            pl.BlockSpec((128, 1), lambda i: (i, 0)),
            pl.BlockSpec((1, D), lambda i: (0, 0)),
            pl.BlockSpec((1, D), lambda i: (0, 0)),
            pl.BlockSpec((1, D), lambda i: (0, 0)),
            pl.BlockSpec((D, D), lambda i: (0, 0)),
            pl.BlockSpec((1, D), lambda i: (0, 0)),
        ],
        out_specs=[
            pl.BlockSpec((128, D), lambda i: (i, 0)),
            pl.BlockSpec((128, D), lambda i: (i, 0)),
        ],
        out_shape=[
            jax.ShapeDtypeStruct((NPAD, D), jnp.float32),
            jax.ShapeDtypeStruct((NPAD, D), jnp.float32),
        ],
    )(s, hp, dinv, b, g, be, wfc_pad, bfc_pad)


def kernel(x, edge_index, W1, b1, g1, be1, W2, b2, g2, be2, W3, b3, g3, be3,
           Wfc, bfc):
    f32 = jnp.float32
    xpad = jnp.zeros((NPAD, D), f32).at[:N].set(x)
    pad_idx = jnp.full((EPAD - E,), NPAD - 1, jnp.int32)
    src = jnp.concatenate([edge_index[0], pad_idx])
    dst = jnp.concatenate([edge_index[1], pad_idx])

    zero128 = jnp.zeros((128, D), f32)
    ones = jnp.ones((NPAD, D), f32)
    wfc_pad = jnp.zeros((D, D), f32).at[:, :C].set(Wfc)
    bfc_pad = jnp.zeros((1, D), f32).at[0, :C].set(bfc)

    scatter = _make_scatter()

    deg_parts = scatter(ones, src, dst, zero128)
    dinv = _dinv_call(deg_parts)

    hp = _mm_scale_call(xpad, W1, dinv)
    s = scatter(hp, src, dst, zero128)
    hp = _combine_mm_call(s, hp, dinv, b1.reshape(1, D), g1.reshape(1, D),
                          be1.reshape(1, D), W2)
    s = scatter(hp, src, dst, zero128)
    hp = _combine_mm_call(s, hp, dinv, b2.reshape(1, D), g2.reshape(1, D),
                          be2.reshape(1, D), W3)
    s = scatter(hp, src, dst, zero128)
    logp, z = _final_call(s, hp, dinv, b3.reshape(1, D), g3.reshape(1, D),
                          be3.reshape(1, D), wfc_pad, bfc_pad)

    return (logp[:N, :C], z[:N, :C])
